# baseline (speedup 1.0000x reference)
import jax
import jax.numpy as jnp
from jax import lax
from jax.experimental import pallas as pl
from jax.experimental.pallas import tpu as pltpu

MEM_SIZE = 16384
KEY_DIM = 128
N_HW = 256
BS = 8
TAU = 0.8
ALPHA = 0.1
N_CLASSES = 10

W = 1024
NT = MEM_SIZE // W

NEG_INF = float("-inf")
INF = float("inf")


def _rowmax_argmax(x, col):
    m = jnp.max(x, axis=1, keepdims=True)
    a = jnp.argmax(x, axis=1).astype(jnp.int32)[:, None] + col[0, 0]
    return m, a


def _normalize_rows(x):
    n = jnp.sqrt(jnp.sum(x * x, axis=-1, keepdims=True))
    return x / jnp.maximum(n, 1e-12)


def _dot(a, b, dims):
    return lax.dot_general(a, b, (dims, ((), ())),
                           preferred_element_type=jnp.float32,
                           precision=lax.Precision.DEFAULT)


def _dot_bf16(a, b, dims):
    return lax.dot_general(a.astype(jnp.bfloat16), b.astype(jnp.bfloat16),
                           (dims, ((), ())),
                           preferred_element_type=jnp.float32)


def _onehot_scatter(mask_f32, rows_f32, dims):
    p0 = rows_f32.astype(jnp.bfloat16)
    r1 = rows_f32 - p0.astype(jnp.float32)
    p1 = r1.astype(jnp.bfloat16)
    p2 = (r1 - p1.astype(jnp.float32)).astype(jnp.bfloat16)
    m = mask_f32.astype(jnp.bfloat16)
    out = lax.dot_general(m, p0, (dims, ((), ())),
                          preferred_element_type=jnp.float32)
    out = out + lax.dot_general(m, p1, (dims, ((), ())),
                                preferred_element_type=jnp.float32)
    return out + lax.dot_general(m, p2, (dims, ((), ())),
                                 preferred_element_type=jnp.float32)


def _any_in_tile(idx, t):
    hit = (idx >= t * W) & (idx < (t + 1) * W)
    return jnp.max(hit.astype(jnp.float32)) > 0.5


def _memory_kernel(tgt_ref, q_ref, keys_in_ref, vals_in_ref, ages_in_ref,
                   loss_ref, fetched_ref, keys_ref, vals_ref, ages_ref):
    keys_ref[:] = keys_in_ref[:]
    vals_ref[:] = vals_in_ref[:]
    ages_ref[:] = ages_in_ref[:]

    jj = lax.broadcasted_iota(jnp.int32, (N_HW, 1), 0)
    wa = lax.broadcasted_iota(jnp.int32, (W, W), 0)
    wb = lax.broadcasted_iota(jnp.int32, (W, W), 1)
    tri_incl = (wa <= wb).astype(jnp.float32)
    ones_j = jnp.ones((N_HW, 1), jnp.float32)
    col_w = lax.broadcasted_iota(jnp.int32, (1, W), 1)
    zk = jnp.zeros((N_HW, KEY_DIM), jnp.float32)
    zi = jnp.zeros((N_HW, 1), jnp.int32)
    zf = jnp.zeros((N_HW, 1), jnp.float32)
    fneg = jnp.full((N_HW, 1), NEG_INF)

    def keys_tile(t):
        return keys_ref[pl.ds(t * W, W), :]

    def body(i, loss):
        tgt = tgt_ref[i]
        qn = _normalize_rows(q_ref[i])
        has_c = jnp.sum((vals_ref[:] == tgt).astype(jnp.float32)) > 0.0

        def nearest_upd(t, sims_t, col_t, near_m, near_a, near_v):
            tm, ta = _rowmax_argmax(sims_t, col_t)
            tile_nv = jnp.sum(
                jnp.where(ta == col_t, vals_ref[t].astype(jnp.float32), 0.0),
                axis=1, keepdims=True)
            upd = tm > near_m
            return (jnp.where(upd, tm, near_m), jnp.where(upd, ta, near_a),
                    jnp.where(upd, tile_nv, near_v))

        def pass_a_full(t, c):
            near_m, near_a, near_v, v1, i1, v2, i2, neg_max = c
            col_t = col_w + t * W
            sims_t = _dot(qn, keys_tile(t), ((1,), (1,)))
            cmf_t = (vals_ref[t] == tgt).astype(jnp.float32)
            near_m, near_a, near_v = nearest_upd(
                t, sims_t, col_t, near_m, near_a, near_v)
            pos_t = sims_t * cmf_t
            neg_max = jnp.maximum(
                neg_max, jnp.max(sims_t * (1.0 - cmf_t), axis=1, keepdims=True))
            tv1, ti1 = _rowmax_argmax(pos_t, col_t)
            pos_mt = jnp.where(col_t == ti1, NEG_INF, pos_t)
            tv2, ti2 = _rowmax_argmax(pos_mt, col_t)
            take1 = tv1 > v1
            nv2 = jnp.where(take1, jnp.where(tv2 > v1, tv2, v1),
                            jnp.where(tv1 > v2, tv1, v2))
            ni2 = jnp.where(take1, jnp.where(tv2 > v1, ti2, i1),
                            jnp.where(tv1 > v2, ti1, i2))
            v1 = jnp.where(take1, tv1, v1)
            i1 = jnp.where(take1, ti1, i1)
            return near_m, near_a, near_v, v1, i1, nv2, ni2, neg_max

        def pass_a_light(t, c):
            near_m, near_a, near_v = c
            col_t = col_w + t * W
            sims_t = _dot(qn, keys_tile(t), ((1,), (1,)))
            return nearest_upd(t, sims_t, col_t, near_m, near_a, near_v)

        def a_full(_):
            return lax.fori_loop(0, NT, pass_a_full,
                                 (fneg, zi, zf, fneg, zi, fneg, zi, fneg),
                                 unroll=2)

        def a_light(_):
            near_m, near_a, near_v = lax.fori_loop(
                0, NT, pass_a_light, (fneg, zi, zf), unroll=2)
            return (near_m, near_a, near_v, zf, zi, zf, zi + 1, near_m)

        (_nm, near_a, near_v, pos_max, pos_arg, v2, s_idx,
         neg_max) = lax.cond(has_c, a_full, a_light, None)

        best_v = jnp.sum((near_v == -1.0).astype(jnp.float32))
        best_c = jnp.int32(0)
        for c in range(1, N_CLASSES + 2):
            cnt = jnp.sum((near_v == float(c - 1)).astype(jnp.float32))
            take = cnt > best_v
            best_v = jnp.where(take, cnt, best_v)
            best_c = jnp.where(take, jnp.int32(c), best_c)
        fetched_ref[i] = best_c - 1

        eff_pos = jnp.where(has_c, pos_max, 0.0)
        loss = loss + jnp.mean(jax.nn.relu(neg_max - eff_pos + ALPHA))

        merge_mask = (pos_max > 0.1) & ((pos_max - v2) < (1.0 - TAU) / 2.0)
        do_merge = jnp.max(merge_mask.astype(jnp.float32)) > 0.5

        def with_merge(_):
            def pass_b(t, c):
                def work(c):
                    keys_n, keys_s, lw1_at = c
                    col_t = col_w + t * W
                    kt = keys_tile(t)
                    g1 = col_t == pos_arg
                    keys_n = keys_n + _onehot_scatter(
                        g1.astype(jnp.float32), kt, ((1,), (0,)))
                    keys_s = keys_s + _onehot_scatter(
                        (col_t == s_idx).astype(jnp.float32), kt, ((1,), (0,)))
                    lastw = jnp.max(jnp.where(g1, jj, -1),
                                    axis=0, keepdims=True)
                    lw1_at = lw1_at + jnp.sum(jnp.where(g1, lastw, 0),
                                              axis=1, keepdims=True)
                    return keys_n, keys_s, lw1_at

                return lax.cond(
                    _any_in_tile(pos_arg, t) | _any_in_tile(s_idx, t),
                    work, lambda c: c, c)

            keys_n, keys_s, lw1_at = lax.fori_loop(0, NT, pass_b, (zk, zk, zi))
            merged = _normalize_rows((keys_n + keys_s) * 0.5)
            win1 = (lw1_at == jj).astype(jnp.float32)

            def pass_c(t, c):
                def work(c):
                    col_t = col_w + t * W
                    g1f = (col_t == pos_arg).astype(jnp.float32)
                    w1 = g1f * win1
                    wm1 = _dot_bf16(w1, ones_j, ((0,), (0,)))
                    scat1 = _onehot_scatter(w1, merged, ((0,), (0,)))
                    keys_ref[pl.ds(t * W, W), :] = jnp.where(
                        wm1 > 0.5, scat1, keys_tile(t))
                    g2f = (col_t == s_idx).astype(jnp.float32)
                    sm = jnp.max(g2f, axis=0, keepdims=True) > 0.5
                    vals_ref[t] = jnp.where(sm, -1, vals_ref[t])
                    ages_ref[t] = jnp.where(sm, INF, ages_ref[t])
                    return c

                return lax.cond(
                    _any_in_tile(pos_arg, t) | _any_in_tile(s_idx, t),
                    work, lambda c: c, c)

            lax.fori_loop(0, NT, pass_c, jnp.int32(0))
            return jnp.int32(0)

        lax.cond(do_merge, with_merge, lambda _: jnp.int32(0), None)

        cl = pos_max > TAU
        any_cl = jnp.max(cl.astype(jnp.float32)) > 0.5

        def with_cl(_):
            def pass_npk(t, npk):
                def work(npk):
                    col_t = col_w + t * W
                    g1f = (col_t == pos_arg).astype(jnp.float32)
                    return npk + _onehot_scatter(g1f, keys_tile(t),
                                                 ((1,), (0,)))

                return lax.cond(_any_in_tile(pos_arg, t), work,
                                lambda npk: npk, npk)

            npk = lax.fori_loop(0, NT, pass_npk, zk)
            return _normalize_rows((npk + qn) * 0.5)

        kcu = lax.cond(any_cl, with_cl, lambda _: zk, None)

        def pass_d0(t, c):
            def work(c):
                off, oldest = c
                col_t = col_w + t * W
                m_t = (ages_ref[t] == INF).astype(jnp.float32)
                rank_t = (_dot_bf16(m_t, tri_incl, ((1,), (0,)))
                          + off).astype(jnp.int32)
                sel = (rank_t == (jj + 1)) & (m_t > 0.5)
                oldest = oldest + jnp.sum(jnp.where(sel, col_t, 0),
                                          axis=1, keepdims=True)
                return off + jnp.sum(m_t), oldest

            return lax.cond(c[0] < float(N_HW), work, lambda c: c, c)

        _off, oldest = lax.fori_loop(0, NT, pass_d0, (jnp.float32(0.0), zi))

        ages_ref[:] = jnp.where(vals_ref[:] == tgt, ages_ref[:] + 1.0,
                                ages_ref[:])

        upd_idx = jnp.where(cl, pos_arg, oldest)
        upd_keys = jnp.where(cl, kcu, qn)

        def cl_win(_):
            def pass_d(t, lwu_at):
                def work(lwu_at):
                    col_t = col_w + t * W
                    u = col_t == upd_idx
                    lastw = jnp.max(jnp.where(u, jj, -1),
                                    axis=0, keepdims=True)
                    return lwu_at + jnp.sum(jnp.where(u, lastw, 0),
                                            axis=1, keepdims=True)

                return lax.cond(_any_in_tile(upd_idx, t), work,
                                lambda x: x, lwu_at)

            lwu_at = lax.fori_loop(0, NT, pass_d, zi)
            return (lwu_at == jj).astype(jnp.float32)

        win = lax.cond(any_cl, cl_win, lambda _: zf + 1.0, None)

        def pass_e(t, c):
            def work(c):
                col_t = col_w + t * W
                uf = (col_t == upd_idx).astype(jnp.float32)
                wu = uf * win
                wmu = _dot_bf16(wu, ones_j, ((0,), (0,)))
                scatu = _onehot_scatter(wu, upd_keys, ((0,), (0,)))
                keys_ref[pl.ds(t * W, W), :] = jnp.where(
                    wmu > 0.5, scatu, keys_tile(t))
                um = jnp.max(uf, axis=0, keepdims=True) > 0.5
                vals_ref[t] = jnp.where(um, tgt, vals_ref[t])
                ages_ref[t] = jnp.where(um, 1.0, ages_ref[t])
                return c

            return lax.cond(_any_in_tile(upd_idx, t), work, lambda c: c, c)

        lax.fori_loop(0, NT, pass_e, jnp.int32(0))
        return loss

    loss = lax.fori_loop(0, BS, body, jnp.float32(0.0))
    loss_ref[0] = loss


@jax.jit
def kernel(queries, targets, m_keys, m_vals, m_ages, age_noise):
    del age_noise
    q = jnp.transpose(queries.reshape(BS, KEY_DIM, N_HW), (0, 2, 1))
    vals3d = m_vals.reshape(NT, 1, W)
    ages3d = m_ages.reshape(NT, 1, W)
    loss, fetched = pl.pallas_call(
        _memory_kernel,
        out_shape=(
            jax.ShapeDtypeStruct((1,), jnp.float32),
            jax.ShapeDtypeStruct((BS,), jnp.int32),
        ),
        in_specs=[
            pl.BlockSpec(memory_space=pltpu.SMEM),
            pl.BlockSpec(memory_space=pltpu.VMEM),
            pl.BlockSpec(memory_space=pltpu.VMEM),
            pl.BlockSpec(memory_space=pltpu.VMEM),
            pl.BlockSpec(memory_space=pltpu.VMEM),
        ],
        out_specs=(
            pl.BlockSpec(memory_space=pltpu.SMEM),
            pl.BlockSpec(memory_space=pltpu.SMEM),
        ),
        scratch_shapes=[
            pltpu.VMEM((MEM_SIZE, KEY_DIM), jnp.float32),
            pltpu.VMEM((NT, 1, W), jnp.int32),
            pltpu.VMEM((NT, 1, W), jnp.float32),
        ],
        compiler_params=pltpu.CompilerParams(
            vmem_limit_bytes=63 * 1024 * 1024,
        ),
    )(targets, q, m_keys, vals3d, ages3d)
    return loss.reshape(()), fetched


# speedup vs baseline: 1.4125x; 1.4125x over previous
import jax
import jax.numpy as jnp
from jax import lax
from jax.experimental import pallas as pl
from jax.experimental.pallas import tpu as pltpu

MEM_SIZE = 16384
KEY_DIM = 128
N_HW = 256
BS = 8
TAU = 0.8
ALPHA = 0.1
N_CLASSES = 10

W = 1024
NT = MEM_SIZE // W

NEG_INF = float("-inf")
INF = float("inf")


def _rowmax_argmax(x, col):
    m = jnp.max(x, axis=1, keepdims=True)
    a = jnp.min(jnp.where(x == m, col, 1 << 30), axis=1, keepdims=True)
    return m, a


def _normalize_rows(x):
    n = jnp.sqrt(jnp.sum(x * x, axis=-1, keepdims=True))
    return x / jnp.maximum(n, 1e-12)


def _dot(a, b, dims):
    return lax.dot_general(a, b, (dims, ((), ())),
                           preferred_element_type=jnp.float32,
                           precision=lax.Precision.DEFAULT)


def _dot_bf16(a, b, dims):
    return lax.dot_general(a.astype(jnp.bfloat16), b.astype(jnp.bfloat16),
                           (dims, ((), ())),
                           preferred_element_type=jnp.float32)


def _onehot_scatter(mask_f32, rows_f32, dims):
    p0 = rows_f32.astype(jnp.bfloat16)
    r1 = rows_f32 - p0.astype(jnp.float32)
    p1 = r1.astype(jnp.bfloat16)
    p2 = (r1 - p1.astype(jnp.float32)).astype(jnp.bfloat16)
    m = mask_f32.astype(jnp.bfloat16)
    out = lax.dot_general(m, p0, (dims, ((), ())),
                          preferred_element_type=jnp.float32)
    out = out + lax.dot_general(m, p1, (dims, ((), ())),
                                preferred_element_type=jnp.float32)
    return out + lax.dot_general(m, p2, (dims, ((), ())),
                                 preferred_element_type=jnp.float32)


def _any_in_tile(idx, t):
    hit = (idx >= t * W) & (idx < (t + 1) * W)
    return jnp.max(hit.astype(jnp.float32)) > 0.5


def _memory_kernel(tgt_ref, q_ref, keys_in_ref, vals_in_ref, ages_in_ref,
                   loss_ref, fetched_ref, keys_ref, vals_ref, ages_ref):
    keys_ref[:] = keys_in_ref[:]
    vals_ref[:] = vals_in_ref[:]
    ages_ref[:] = ages_in_ref[:]

    jj = lax.broadcasted_iota(jnp.int32, (N_HW, 1), 0)
    wa = lax.broadcasted_iota(jnp.int32, (W, W), 0)
    wb = lax.broadcasted_iota(jnp.int32, (W, W), 1)
    tri_incl = (wa <= wb).astype(jnp.float32)
    ones_j = jnp.ones((N_HW, 1), jnp.float32)
    col_w = lax.broadcasted_iota(jnp.int32, (1, W), 1)
    zk = jnp.zeros((N_HW, KEY_DIM), jnp.float32)
    zi = jnp.zeros((N_HW, 1), jnp.int32)
    zf = jnp.zeros((N_HW, 1), jnp.float32)
    fneg = jnp.full((N_HW, 1), NEG_INF)

    def keys_tile(t):
        return keys_ref[pl.ds(t * W, W), :]

    def body(i, loss):
        tgt = tgt_ref[i]
        qn = _normalize_rows(q_ref[i])
        has_c = jnp.sum((vals_ref[:] == tgt).astype(jnp.float32)) > 0.0

        def nearest_upd(t, sims_t, col_t, near_m, near_a, near_v):
            tm, ta = _rowmax_argmax(sims_t, col_t)
            tile_nv = jnp.sum(
                jnp.where(ta == col_t, vals_ref[t].astype(jnp.float32), 0.0),
                axis=1, keepdims=True)
            upd = tm > near_m
            return (jnp.where(upd, tm, near_m), jnp.where(upd, ta, near_a),
                    jnp.where(upd, tile_nv, near_v))

        def pass_a_full(t, c):
            near_m, near_a, near_v, v1, i1, v2, i2, neg_max = c
            col_t = col_w + t * W
            sims_t = _dot(qn, keys_tile(t), ((1,), (1,)))
            cmf_t = (vals_ref[t] == tgt).astype(jnp.float32)
            near_m, near_a, near_v = nearest_upd(
                t, sims_t, col_t, near_m, near_a, near_v)
            pos_t = sims_t * cmf_t
            neg_max = jnp.maximum(
                neg_max, jnp.max(sims_t * (1.0 - cmf_t), axis=1, keepdims=True))
            tv1, ti1 = _rowmax_argmax(pos_t, col_t)
            pos_mt = jnp.where(col_t == ti1, NEG_INF, pos_t)
            tv2, ti2 = _rowmax_argmax(pos_mt, col_t)
            take1 = tv1 > v1
            nv2 = jnp.where(take1, jnp.where(tv2 > v1, tv2, v1),
                            jnp.where(tv1 > v2, tv1, v2))
            ni2 = jnp.where(take1, jnp.where(tv2 > v1, ti2, i1),
                            jnp.where(tv1 > v2, ti1, i2))
            v1 = jnp.where(take1, tv1, v1)
            i1 = jnp.where(take1, ti1, i1)
            return near_m, near_a, near_v, v1, i1, nv2, ni2, neg_max

        def pass_a_light(t, c):
            near_m, near_a, near_v = c
            col_t = col_w + t * W
            sims_t = _dot(qn, keys_tile(t), ((1,), (1,)))
            return nearest_upd(t, sims_t, col_t, near_m, near_a, near_v)

        def a_full(_):
            return lax.fori_loop(0, NT, pass_a_full,
                                 (fneg, zi, zf, fneg, zi, fneg, zi, fneg),
                                 unroll=4)

        def a_light(_):
            near_m, near_a, near_v = lax.fori_loop(
                0, NT, pass_a_light, (fneg, zi, zf), unroll=4)
            return (near_m, near_a, near_v, zf, zi, zf, zi + 1, near_m)

        (_nm, near_a, near_v, pos_max, pos_arg, v2, s_idx,
         neg_max) = lax.cond(has_c, a_full, a_light, None)

        best_v = jnp.sum((near_v == -1.0).astype(jnp.float32))
        best_c = jnp.int32(0)
        for c in range(1, N_CLASSES + 2):
            cnt = jnp.sum((near_v == float(c - 1)).astype(jnp.float32))
            take = cnt > best_v
            best_v = jnp.where(take, cnt, best_v)
            best_c = jnp.where(take, jnp.int32(c), best_c)
        fetched_ref[i] = best_c - 1

        eff_pos = jnp.where(has_c, pos_max, 0.0)
        loss = loss + jnp.mean(jax.nn.relu(neg_max - eff_pos + ALPHA))

        merge_mask = (pos_max > 0.1) & ((pos_max - v2) < (1.0 - TAU) / 2.0)
        do_merge = jnp.max(merge_mask.astype(jnp.float32)) > 0.5

        def with_merge(_):
            def pass_b(t, c):
                def work(c):
                    keys_n, keys_s, lw1_at = c
                    col_t = col_w + t * W
                    kt = keys_tile(t)
                    g1 = col_t == pos_arg
                    keys_n = keys_n + _onehot_scatter(
                        g1.astype(jnp.float32), kt, ((1,), (0,)))
                    keys_s = keys_s + _onehot_scatter(
                        (col_t == s_idx).astype(jnp.float32), kt, ((1,), (0,)))
                    lastw = jnp.max(jnp.where(g1, jj, -1),
                                    axis=0, keepdims=True)
                    lw1_at = lw1_at + jnp.sum(jnp.where(g1, lastw, 0),
                                              axis=1, keepdims=True)
                    return keys_n, keys_s, lw1_at

                return lax.cond(
                    _any_in_tile(pos_arg, t) | _any_in_tile(s_idx, t),
                    work, lambda c: c, c)

            keys_n, keys_s, lw1_at = lax.fori_loop(0, NT, pass_b, (zk, zk, zi))
            merged = _normalize_rows((keys_n + keys_s) * 0.5)
            win1 = (lw1_at == jj).astype(jnp.float32)

            def pass_c(t, c):
                def work(c):
                    col_t = col_w + t * W
                    g1f = (col_t == pos_arg).astype(jnp.float32)
                    w1 = g1f * win1
                    wm1 = _dot_bf16(w1, ones_j, ((0,), (0,)))
                    scat1 = _onehot_scatter(w1, merged, ((0,), (0,)))
                    keys_ref[pl.ds(t * W, W), :] = jnp.where(
                        wm1 > 0.5, scat1, keys_tile(t))
                    g2f = (col_t == s_idx).astype(jnp.float32)
                    sm = jnp.max(g2f, axis=0, keepdims=True) > 0.5
                    vals_ref[t] = jnp.where(sm, -1, vals_ref[t])
                    ages_ref[t] = jnp.where(sm, INF, ages_ref[t])
                    return c

                return lax.cond(
                    _any_in_tile(pos_arg, t) | _any_in_tile(s_idx, t),
                    work, lambda c: c, c)

            lax.fori_loop(0, NT, pass_c, jnp.int32(0))
            return jnp.int32(0)

        lax.cond(do_merge, with_merge, lambda _: jnp.int32(0), None)

        cl = pos_max > TAU
        any_cl = jnp.max(cl.astype(jnp.float32)) > 0.5

        def with_cl(_):
            def pass_npk(t, npk):
                def work(npk):
                    col_t = col_w + t * W
                    g1f = (col_t == pos_arg).astype(jnp.float32)
                    return npk + _onehot_scatter(g1f, keys_tile(t),
                                                 ((1,), (0,)))

                return lax.cond(_any_in_tile(pos_arg, t), work,
                                lambda npk: npk, npk)

            npk = lax.fori_loop(0, NT, pass_npk, zk)
            return _normalize_rows((npk + qn) * 0.5)

        kcu = lax.cond(any_cl, with_cl, lambda _: zk, None)

        def pass_d0(t, c):
            def work(c):
                off, oldest = c
                col_t = col_w + t * W
                m_t = (ages_ref[t] == INF).astype(jnp.float32)
                rank_t = (_dot_bf16(m_t, tri_incl, ((1,), (0,)))
                          + off).astype(jnp.int32)
                sel = (rank_t == (jj + 1)) & (m_t > 0.5)
                oldest = oldest + jnp.sum(jnp.where(sel, col_t, 0),
                                          axis=1, keepdims=True)
                return off + jnp.sum(m_t), oldest

            return lax.cond(c[0] < float(N_HW), work, lambda c: c, c)

        _off, oldest = lax.fori_loop(0, NT, pass_d0, (jnp.float32(0.0), zi))

        ages_ref[:] = jnp.where(vals_ref[:] == tgt, ages_ref[:] + 1.0,
                                ages_ref[:])

        upd_idx = jnp.where(cl, pos_arg, oldest)
        upd_keys = jnp.where(cl, kcu, qn)

        def cl_win(_):
            def pass_d(t, lwu_at):
                def work(lwu_at):
                    col_t = col_w + t * W
                    u = col_t == upd_idx
                    lastw = jnp.max(jnp.where(u, jj, -1),
                                    axis=0, keepdims=True)
                    return lwu_at + jnp.sum(jnp.where(u, lastw, 0),
                                            axis=1, keepdims=True)

                return lax.cond(_any_in_tile(upd_idx, t), work,
                                lambda x: x, lwu_at)

            lwu_at = lax.fori_loop(0, NT, pass_d, zi)
            return (lwu_at == jj).astype(jnp.float32)

        win = lax.cond(any_cl, cl_win, lambda _: zf + 1.0, None)

        def pass_e(t, c):
            def work(c):
                col_t = col_w + t * W
                uf = (col_t == upd_idx).astype(jnp.float32)
                wu = uf * win
                wmu = _dot_bf16(wu, ones_j, ((0,), (0,)))
                scatu = _onehot_scatter(wu, upd_keys, ((0,), (0,)))
                keys_ref[pl.ds(t * W, W), :] = jnp.where(
                    wmu > 0.5, scatu, keys_tile(t))
                um = jnp.max(uf, axis=0, keepdims=True) > 0.5
                vals_ref[t] = jnp.where(um, tgt, vals_ref[t])
                ages_ref[t] = jnp.where(um, 1.0, ages_ref[t])
                return c

            return lax.cond(_any_in_tile(upd_idx, t), work, lambda c: c, c)

        lax.fori_loop(0, NT, pass_e, jnp.int32(0))
        return loss

    loss = lax.fori_loop(0, BS, body, jnp.float32(0.0))
    loss_ref[0] = loss


@jax.jit
def kernel(queries, targets, m_keys, m_vals, m_ages, age_noise):
    del age_noise
    q = jnp.transpose(queries.reshape(BS, KEY_DIM, N_HW), (0, 2, 1))
    vals3d = m_vals.reshape(NT, 1, W)
    ages3d = m_ages.reshape(NT, 1, W)
    loss, fetched = pl.pallas_call(
        _memory_kernel,
        out_shape=(
            jax.ShapeDtypeStruct((1,), jnp.float32),
            jax.ShapeDtypeStruct((BS,), jnp.int32),
        ),
        in_specs=[
            pl.BlockSpec(memory_space=pltpu.SMEM),
            pl.BlockSpec(memory_space=pltpu.VMEM),
            pl.BlockSpec(memory_space=pltpu.VMEM),
            pl.BlockSpec(memory_space=pltpu.VMEM),
            pl.BlockSpec(memory_space=pltpu.VMEM),
        ],
        out_specs=(
            pl.BlockSpec(memory_space=pltpu.SMEM),
            pl.BlockSpec(memory_space=pltpu.SMEM),
        ),
        scratch_shapes=[
            pltpu.VMEM((MEM_SIZE, KEY_DIM), jnp.float32),
            pltpu.VMEM((NT, 1, W), jnp.int32),
            pltpu.VMEM((NT, 1, W), jnp.float32),
        ],
        compiler_params=pltpu.CompilerParams(
            vmem_limit_bytes=63 * 1024 * 1024,
        ),
    )(targets, q, m_keys, vals3d, ages3d)
    return loss.reshape(()), fetched


# speedup vs baseline: 1.4213x; 1.0062x over previous
import jax
import jax.numpy as jnp
from jax import lax
from jax.experimental import pallas as pl
from jax.experimental.pallas import tpu as pltpu

MEM_SIZE = 16384
KEY_DIM = 128
N_HW = 256
BS = 8
TAU = 0.8
ALPHA = 0.1
N_CLASSES = 10

W = 1024
NT = MEM_SIZE // W

NEG_INF = float("-inf")
INF = float("inf")


def _rowmax_argmax(x, col):
    m = jnp.max(x, axis=1, keepdims=True)
    a = jnp.min(jnp.where(x == m, col, 1 << 30), axis=1, keepdims=True)
    return m, a


def _normalize_rows(x):
    n = jnp.sqrt(jnp.sum(x * x, axis=-1, keepdims=True))
    return x / jnp.maximum(n, 1e-12)


def _dot(a, b, dims):
    return lax.dot_general(a, b, (dims, ((), ())),
                           preferred_element_type=jnp.float32,
                           precision=lax.Precision.DEFAULT)


def _dot_bf16(a, b, dims):
    return lax.dot_general(a.astype(jnp.bfloat16), b.astype(jnp.bfloat16),
                           (dims, ((), ())),
                           preferred_element_type=jnp.float32)


def _onehot_scatter(mask_f32, rows_f32, dims):
    p0 = rows_f32.astype(jnp.bfloat16)
    r1 = rows_f32 - p0.astype(jnp.float32)
    p1 = r1.astype(jnp.bfloat16)
    p2 = (r1 - p1.astype(jnp.float32)).astype(jnp.bfloat16)
    m = mask_f32.astype(jnp.bfloat16)
    out = lax.dot_general(m, p0, (dims, ((), ())),
                          preferred_element_type=jnp.float32)
    out = out + lax.dot_general(m, p1, (dims, ((), ())),
                                preferred_element_type=jnp.float32)
    return out + lax.dot_general(m, p2, (dims, ((), ())),
                                 preferred_element_type=jnp.float32)


def _any_in_tile(idx, t):
    hit = (idx >= t * W) & (idx < (t + 1) * W)
    return jnp.max(hit.astype(jnp.float32)) > 0.5


def _memory_kernel(tgt_ref, q_ref, keys_in_ref, vals_in_ref, ages_in_ref,
                   loss_ref, fetched_ref, keys_ref, vals_ref, ages_ref):
    keys_ref[:] = keys_in_ref[:]
    vals_ref[:] = vals_in_ref[:]
    ages_ref[:] = ages_in_ref[:]

    jj = lax.broadcasted_iota(jnp.int32, (N_HW, 1), 0)
    wa = lax.broadcasted_iota(jnp.int32, (W, W), 0)
    wb = lax.broadcasted_iota(jnp.int32, (W, W), 1)
    tri_incl = (wa <= wb).astype(jnp.float32)
    ones_j = jnp.ones((N_HW, 1), jnp.float32)
    col_w = lax.broadcasted_iota(jnp.int32, (1, W), 1)
    zk = jnp.zeros((N_HW, KEY_DIM), jnp.float32)
    zi = jnp.zeros((N_HW, 1), jnp.int32)
    zf = jnp.zeros((N_HW, 1), jnp.float32)
    fneg = jnp.full((N_HW, 1), NEG_INF)

    def keys_tile(t):
        return keys_ref[pl.ds(t * W, W), :]

    def body(i, loss):
        tgt = tgt_ref[i]
        qn = _normalize_rows(q_ref[i])
        has_c = jnp.sum((vals_ref[:] == tgt).astype(jnp.float32)) > 0.0

        def nearest_upd(t, sims_t, col_t, near_m, near_a, near_v):
            tm, ta = _rowmax_argmax(sims_t, col_t)
            tile_nv = jnp.sum(
                jnp.where(ta == col_t, vals_ref[t].astype(jnp.float32), 0.0),
                axis=1, keepdims=True)
            upd = tm > near_m
            return (jnp.where(upd, tm, near_m), jnp.where(upd, ta, near_a),
                    jnp.where(upd, tile_nv, near_v))

        def pass_a_full(t, c):
            near_m, near_a, near_v, v1, i1, v2, i2, neg_max = c
            col_t = col_w + t * W
            sims_t = _dot(qn, keys_tile(t), ((1,), (1,)))
            cmf_t = (vals_ref[t] == tgt).astype(jnp.float32)
            near_m, near_a, near_v = nearest_upd(
                t, sims_t, col_t, near_m, near_a, near_v)
            pos_t = sims_t * cmf_t
            neg_max = jnp.maximum(
                neg_max, jnp.max(sims_t * (1.0 - cmf_t), axis=1, keepdims=True))
            tv1, ti1 = _rowmax_argmax(pos_t, col_t)
            pos_mt = jnp.where(col_t == ti1, NEG_INF, pos_t)
            tv2, ti2 = _rowmax_argmax(pos_mt, col_t)
            take1 = tv1 > v1
            nv2 = jnp.where(take1, jnp.where(tv2 > v1, tv2, v1),
                            jnp.where(tv1 > v2, tv1, v2))
            ni2 = jnp.where(take1, jnp.where(tv2 > v1, ti2, i1),
                            jnp.where(tv1 > v2, ti1, i2))
            v1 = jnp.where(take1, tv1, v1)
            i1 = jnp.where(take1, ti1, i1)
            return near_m, near_a, near_v, v1, i1, nv2, ni2, neg_max

        def pass_a_light(t, c):
            near_m, near_a, near_v = c
            col_t = col_w + t * W
            sims_t = _dot(qn, keys_tile(t), ((1,), (1,)))
            return nearest_upd(t, sims_t, col_t, near_m, near_a, near_v)

        def a_full(_):
            return lax.fori_loop(0, NT, pass_a_full,
                                 (fneg, zi, zf, fneg, zi, fneg, zi, fneg),
                                 unroll=8)

        def a_light(_):
            near_m, near_a, near_v = lax.fori_loop(
                0, NT, pass_a_light, (fneg, zi, zf), unroll=8)
            return (near_m, near_a, near_v, zf, zi, zf, zi + 1, near_m)

        (_nm, near_a, near_v, pos_max, pos_arg, v2, s_idx,
         neg_max) = lax.cond(has_c, a_full, a_light, None)

        best_v = jnp.sum((near_v == -1.0).astype(jnp.float32))
        best_c = jnp.int32(0)
        for c in range(1, N_CLASSES + 2):
            cnt = jnp.sum((near_v == float(c - 1)).astype(jnp.float32))
            take = cnt > best_v
            best_v = jnp.where(take, cnt, best_v)
            best_c = jnp.where(take, jnp.int32(c), best_c)
        fetched_ref[i] = best_c - 1

        eff_pos = jnp.where(has_c, pos_max, 0.0)
        loss = loss + jnp.mean(jax.nn.relu(neg_max - eff_pos + ALPHA))

        merge_mask = (pos_max > 0.1) & ((pos_max - v2) < (1.0 - TAU) / 2.0)
        do_merge = jnp.max(merge_mask.astype(jnp.float32)) > 0.5

        def with_merge(_):
            def pass_b(t, c):
                def work(c):
                    keys_n, keys_s, lw1_at = c
                    col_t = col_w + t * W
                    kt = keys_tile(t)
                    g1 = col_t == pos_arg
                    keys_n = keys_n + _onehot_scatter(
                        g1.astype(jnp.float32), kt, ((1,), (0,)))
                    keys_s = keys_s + _onehot_scatter(
                        (col_t == s_idx).astype(jnp.float32), kt, ((1,), (0,)))
                    lastw = jnp.max(jnp.where(g1, jj, -1),
                                    axis=0, keepdims=True)
                    lw1_at = lw1_at + jnp.sum(jnp.where(g1, lastw, 0),
                                              axis=1, keepdims=True)
                    return keys_n, keys_s, lw1_at

                return lax.cond(
                    _any_in_tile(pos_arg, t) | _any_in_tile(s_idx, t),
                    work, lambda c: c, c)

            keys_n, keys_s, lw1_at = lax.fori_loop(0, NT, pass_b, (zk, zk, zi))
            merged = _normalize_rows((keys_n + keys_s) * 0.5)
            win1 = (lw1_at == jj).astype(jnp.float32)

            def pass_c(t, c):
                def work(c):
                    col_t = col_w + t * W
                    g1f = (col_t == pos_arg).astype(jnp.float32)
                    w1 = g1f * win1
                    wm1 = _dot_bf16(w1, ones_j, ((0,), (0,)))
                    scat1 = _onehot_scatter(w1, merged, ((0,), (0,)))
                    keys_ref[pl.ds(t * W, W), :] = jnp.where(
                        wm1 > 0.5, scat1, keys_tile(t))
                    g2f = (col_t == s_idx).astype(jnp.float32)
                    sm = jnp.max(g2f, axis=0, keepdims=True) > 0.5
                    vals_ref[t] = jnp.where(sm, -1, vals_ref[t])
                    ages_ref[t] = jnp.where(sm, INF, ages_ref[t])
                    return c

                return lax.cond(
                    _any_in_tile(pos_arg, t) | _any_in_tile(s_idx, t),
                    work, lambda c: c, c)

            lax.fori_loop(0, NT, pass_c, jnp.int32(0))
            return jnp.int32(0)

        lax.cond(do_merge, with_merge, lambda _: jnp.int32(0), None)

        cl = pos_max > TAU
        any_cl = jnp.max(cl.astype(jnp.float32)) > 0.5

        def with_cl(_):
            def pass_npk(t, npk):
                def work(npk):
                    col_t = col_w + t * W
                    g1f = (col_t == pos_arg).astype(jnp.float32)
                    return npk + _onehot_scatter(g1f, keys_tile(t),
                                                 ((1,), (0,)))

                return lax.cond(_any_in_tile(pos_arg, t), work,
                                lambda npk: npk, npk)

            npk = lax.fori_loop(0, NT, pass_npk, zk)
            return _normalize_rows((npk + qn) * 0.5)

        kcu = lax.cond(any_cl, with_cl, lambda _: zk, None)

        def pass_d0(t, c):
            def work(c):
                off, oldest = c
                col_t = col_w + t * W
                m_t = (ages_ref[t] == INF).astype(jnp.float32)
                rank_t = (_dot_bf16(m_t, tri_incl, ((1,), (0,)))
                          + off).astype(jnp.int32)
                sel = (rank_t == (jj + 1)) & (m_t > 0.5)
                oldest = oldest + jnp.sum(jnp.where(sel, col_t, 0),
                                          axis=1, keepdims=True)
                return off + jnp.sum(m_t), oldest

            return lax.cond(c[0] < float(N_HW), work, lambda c: c, c)

        _off, oldest = lax.fori_loop(0, NT, pass_d0, (jnp.float32(0.0), zi))

        ages_ref[:] = jnp.where(vals_ref[:] == tgt, ages_ref[:] + 1.0,
                                ages_ref[:])

        upd_idx = jnp.where(cl, pos_arg, oldest)
        upd_keys = jnp.where(cl, kcu, qn)

        def cl_win(_):
            def pass_d(t, lwu_at):
                def work(lwu_at):
                    col_t = col_w + t * W
                    u = col_t == upd_idx
                    lastw = jnp.max(jnp.where(u, jj, -1),
                                    axis=0, keepdims=True)
                    return lwu_at + jnp.sum(jnp.where(u, lastw, 0),
                                            axis=1, keepdims=True)

                return lax.cond(_any_in_tile(upd_idx, t), work,
                                lambda x: x, lwu_at)

            lwu_at = lax.fori_loop(0, NT, pass_d, zi)
            return (lwu_at == jj).astype(jnp.float32)

        win = lax.cond(any_cl, cl_win, lambda _: zf + 1.0, None)

        def pass_e(t, c):
            def work(c):
                col_t = col_w + t * W
                uf = (col_t == upd_idx).astype(jnp.float32)
                wu = uf * win
                wmu = _dot_bf16(wu, ones_j, ((0,), (0,)))
                scatu = _onehot_scatter(wu, upd_keys, ((0,), (0,)))
                keys_ref[pl.ds(t * W, W), :] = jnp.where(
                    wmu > 0.5, scatu, keys_tile(t))
                um = jnp.max(uf, axis=0, keepdims=True) > 0.5
                vals_ref[t] = jnp.where(um, tgt, vals_ref[t])
                ages_ref[t] = jnp.where(um, 1.0, ages_ref[t])
                return c

            return lax.cond(_any_in_tile(upd_idx, t), work, lambda c: c, c)

        lax.fori_loop(0, NT, pass_e, jnp.int32(0))
        return loss

    loss = lax.fori_loop(0, BS, body, jnp.float32(0.0))
    loss_ref[0] = loss


@jax.jit
def kernel(queries, targets, m_keys, m_vals, m_ages, age_noise):
    del age_noise
    q = jnp.transpose(queries.reshape(BS, KEY_DIM, N_HW), (0, 2, 1))
    vals3d = m_vals.reshape(NT, 1, W)
    ages3d = m_ages.reshape(NT, 1, W)
    loss, fetched = pl.pallas_call(
        _memory_kernel,
        out_shape=(
            jax.ShapeDtypeStruct((1,), jnp.float32),
            jax.ShapeDtypeStruct((BS,), jnp.int32),
        ),
        in_specs=[
            pl.BlockSpec(memory_space=pltpu.SMEM),
            pl.BlockSpec(memory_space=pltpu.VMEM),
            pl.BlockSpec(memory_space=pltpu.VMEM),
            pl.BlockSpec(memory_space=pltpu.VMEM),
            pl.BlockSpec(memory_space=pltpu.VMEM),
        ],
        out_specs=(
            pl.BlockSpec(memory_space=pltpu.SMEM),
            pl.BlockSpec(memory_space=pltpu.SMEM),
        ),
        scratch_shapes=[
            pltpu.VMEM((MEM_SIZE, KEY_DIM), jnp.float32),
            pltpu.VMEM((NT, 1, W), jnp.int32),
            pltpu.VMEM((NT, 1, W), jnp.float32),
        ],
        compiler_params=pltpu.CompilerParams(
            vmem_limit_bytes=63 * 1024 * 1024,
        ),
    )(targets, q, m_keys, vals3d, ages3d)
    return loss.reshape(()), fetched
